# baseline (speedup 1.0000x reference)
import functools

import jax
import jax.numpy as jnp
from jax import lax
from jax.experimental import pallas as pl
from jax.experimental.pallas import tpu as pltpu
from jax.experimental.pallas import tpu_sc as plsc

NC = 2
NS = 16
L = 16
D = 64
PW = 128
NW = NC * NS

CHUNK = 128


def _make_kernel(batch):
    bpw = batch // NW
    n_chunks = bpw // CHUNK
    mesh = plsc.VectorSubcoreMesh(core_axis_name="c", subcore_axis_name="s")

    @functools.partial(
        pl.kernel,
        mesh=mesh,
        out_type=jax.ShapeDtypeStruct((batch,), jnp.float32),
        compiler_params=pltpu.CompilerParams(use_tc_tiling_on_sc=True),
        scratch_types=[
            pltpu.VMEM((bpw,), jnp.int32),
            pltpu.VMEM((bpw,), jnp.int32),
            pltpu.VMEM((CHUNK,), jnp.int32),
            pltpu.VMEM((CHUNK,), jnp.int32),
            pltpu.VMEM((CHUNK, PW), jnp.float32),
            pltpu.VMEM((CHUNK, PW), jnp.float32),
            pltpu.VMEM((bpw,), jnp.float32),
            pltpu.SemaphoreType.DMA,
            pltpu.SemaphoreType.DMA,
        ],
    )
    def k(user_hbm, item_hbm, ut_hbm, it_hbm, out_hbm,
          uidx_v, iidx_v, urow_v, irow_v, urows_v, irows_v, out_v,
          sem_u, sem_i):
        wid = lax.axis_index("s") * NC + lax.axis_index("c")
        base = wid * bpw

        pltpu.sync_copy(user_hbm.at[pl.ds(base, bpw)], uidx_v)
        pltpu.sync_copy(item_hbm.at[pl.ds(base, bpw)], iidx_v)

        lanes = lax.iota(jnp.int32, L)

        def perm_xor(v, s):
            return v.at[lanes ^ s].get(mode="promise_in_bounds")

        def chunk_body(c, _):
            c0 = c * CHUNK
            for v in range(CHUNK // L):
                urow_v[pl.ds(v * L, L)] = (
                    uidx_v[pl.ds(c0 + v * L, L)] >> 1)
                irow_v[pl.ds(v * L, L)] = (
                    iidx_v[pl.ds(c0 + v * L, L)] >> 1)
            cpu = pltpu.async_copy(ut_hbm.at[urow_v], urows_v, sem_u)
            cpi = pltpu.async_copy(it_hbm.at[irow_v], irows_v, sem_i)
            cpu.wait()
            cpi.wait()

            def group(g, _):
                r0 = g * L
                usel = (uidx_v[pl.ds(c0 + r0, L)] & 1) * D
                isel = (iidx_v[pl.ds(c0 + r0, L)] & 1) * D
                accs = []
                for r in range(L):
                    uo = usel[r]
                    io = isel[r]
                    acc = (urows_v[r0 + r, pl.ds(uo, L)] *
                           irows_v[r0 + r, pl.ds(io, L)])
                    for jc in range(1, D // L):
                        acc = acc + (
                            urows_v[r0 + r, pl.ds(uo + jc * L, L)] *
                            irows_v[r0 + r, pl.ds(io + jc * L, L)])
                    accs.append(acc)
                s = 1
                while len(accs) > 1:
                    lo_mask = (lanes & s) == 0
                    nxt = []
                    for i in range(0, len(accs), 2):
                        a, b = accs[i], accs[i + 1]
                        merged = (jnp.where(lo_mask, a, perm_xor(b, s)) +
                                  jnp.where(lo_mask, perm_xor(a, s), b))
                        nxt.append(merged)
                    accs = nxt
                    s *= 2
                out_v[pl.ds(c0 + r0, L)] = accs[0]
                return _

            lax.fori_loop(0, CHUNK // L, group, None)
            return _

        lax.fori_loop(0, n_chunks, chunk_body, None)

        pltpu.sync_copy(out_v, out_hbm.at[pl.ds(base, bpw)])

    return k


def kernel(user, item, user_table, item_table):
    n_rows, dim = user_table.shape
    k = _make_kernel(user.shape[0])
    ut2 = user_table.reshape(n_rows * dim // PW, PW)
    it2 = jnp.concatenate([item_table[0::2], item_table[1::2]], axis=1)
    ut2, it2 = jax.lax.optimization_barrier((ut2, it2))
    return k(user, item, ut2, it2)


# speedup vs baseline: 8.4552x; 8.4552x over previous
import functools

import jax
import jax.numpy as jnp
from jax import lax
from jax.experimental import pallas as pl
from jax.experimental.pallas import tpu as pltpu
from jax.experimental.pallas import tpu_sc as plsc

NC = 2
NS = 16
L = 16
D = 64
NW = NC * NS

IDX_CHUNK = 128


def _make_kernel(batch):
    bpw = batch // NW
    n_chunks = bpw // IDX_CHUNK
    n_groups = bpw // L
    mesh = plsc.VectorSubcoreMesh(core_axis_name="c", subcore_axis_name="s")

    @functools.partial(
        pl.kernel,
        mesh=mesh,
        out_type=jax.ShapeDtypeStruct((batch,), jnp.float32),
        compiler_params=pltpu.CompilerParams(use_tc_tiling_on_sc=False),
        scratch_types=[
            pltpu.VMEM((n_chunks, IDX_CHUNK), jnp.int32),
            pltpu.VMEM((n_chunks, IDX_CHUNK), jnp.int32),
            pltpu.VMEM((bpw, D), jnp.float32),
            pltpu.VMEM((bpw, D), jnp.float32),
            pltpu.VMEM((bpw,), jnp.float32),
            pltpu.SemaphoreType.DMA,
            pltpu.SemaphoreType.DMA,
        ],
    )
    def k(user_hbm, item_hbm, ut_hbm, it_hbm, out_hbm,
          uidx_v, iidx_v, urows_v, irows_v, out_v, sem_u, sem_i):
        wid = lax.axis_index("s") * NC + lax.axis_index("c")
        base = wid * bpw

        for c in range(n_chunks):
            pltpu.sync_copy(user_hbm.at[pl.ds(base + c * IDX_CHUNK, IDX_CHUNK)],
                            uidx_v.at[c])
            pltpu.sync_copy(item_hbm.at[pl.ds(base + c * IDX_CHUNK, IDX_CHUNK)],
                            iidx_v.at[c])

        copies = []
        for c in range(n_chunks):
            copies.append(pltpu.async_copy(
                ut_hbm.at[uidx_v.at[c]],
                urows_v.at[pl.ds(c * IDX_CHUNK, IDX_CHUNK)], sem_u))
            copies.append(pltpu.async_copy(
                it_hbm.at[iidx_v.at[c]],
                irows_v.at[pl.ds(c * IDX_CHUNK, IDX_CHUNK)], sem_i))
        for cp in copies:
            cp.wait()

        lanes = lax.iota(jnp.int32, L)

        def perm_xor(v, s):
            return v.at[lanes ^ s].get(mode="promise_in_bounds")

        def group(g, _):
            row0 = g * L
            accs = []
            for r in range(L):
                acc = (urows_v[row0 + r, pl.ds(0, L)] *
                       irows_v[row0 + r, pl.ds(0, L)])
                for jc in range(1, D // L):
                    acc = acc + (urows_v[row0 + r, pl.ds(jc * L, L)] *
                                 irows_v[row0 + r, pl.ds(jc * L, L)])
                accs.append(acc)
            s = 1
            while len(accs) > 1:
                lo_mask = (lanes & s) == 0
                nxt = []
                for i in range(0, len(accs), 2):
                    a, b = accs[i], accs[i + 1]
                    merged = (jnp.where(lo_mask, a, perm_xor(b, s)) +
                              jnp.where(lo_mask, perm_xor(a, s), b))
                    nxt.append(merged)
                accs = nxt
                s *= 2
            out_v[pl.ds(row0, L)] = accs[0]
            return _

        lax.fori_loop(0, n_groups, group, None)

        pltpu.sync_copy(out_v, out_hbm.at[pl.ds(base, bpw)])

    return k


def kernel(user, item, user_table, item_table):
    k = _make_kernel(user.shape[0])
    return k(user, item, user_table, item_table)
